# Initial kernel scaffold; baseline (speedup 1.0000x reference)
import jax
import jax.numpy as jnp
from jax.experimental import pallas as pl


def kernel(x, w1, w2):
    raise NotImplementedError("write your pallas kernel here")


# speedup vs baseline: 2.2715x; 2.2715x over previous
import jax
import jax.numpy as jnp
from jax.experimental import pallas as pl
from jax.experimental.pallas import tpu as pltpu

IN_DIM = 768
OUT_DIM = 768
N_PARALLEL = 4
N_EXP = 8
BT = 1024


def _moe_body(x_ref, w1_ref, w2_ref, o_ref):
    x = x_ref[...]
    h = jnp.dot(x, w1_ref[...], preferred_element_type=jnp.float32)
    z_parts = []
    for p in range(N_PARALLEL):
        hp = h[:, p * N_EXP:(p + 1) * N_EXP]
        m = jnp.max(hp, axis=1, keepdims=True)
        eq = hp == m
        lane = jax.lax.broadcasted_iota(jnp.int32, hp.shape, 1)
        cand = jnp.where(eq, lane, N_EXP)
        argmax = jnp.min(cand, axis=1, keepdims=True)
        z_parts.append(jnp.where(lane == argmax, hp, 0.0))
    z = jnp.concatenate(z_parts, axis=1)
    o_ref[...] = jnp.dot(z, w2_ref[...], preferred_element_type=jnp.float32)


def kernel(x, w1, w2):
    s = x.shape
    xf = x.reshape(-1, IN_DIM)
    T = xf.shape[0]
    w1f = w1.reshape(IN_DIM, N_PARALLEL * N_EXP)
    w2f = w2.reshape(N_PARALLEL * N_EXP, OUT_DIM)
    out = pl.pallas_call(
        _moe_body,
        grid=(T // BT,),
        in_specs=[
            pl.BlockSpec((BT, IN_DIM), lambda i: (i, 0)),
            pl.BlockSpec((IN_DIM, N_PARALLEL * N_EXP), lambda i: (0, 0)),
            pl.BlockSpec((N_PARALLEL * N_EXP, OUT_DIM), lambda i: (0, 0)),
        ],
        out_specs=pl.BlockSpec((BT, OUT_DIM), lambda i: (i, 0)),
        out_shape=jax.ShapeDtypeStruct((T, OUT_DIM), jnp.float32),
        compiler_params=pltpu.CompilerParams(
            dimension_semantics=("parallel",),
        ),
    )(xf, w1f, w2f)
    return out.reshape(s[:-1] + (OUT_DIM,))
